# baseline (speedup 1.0000x reference)
import jax
import jax.numpy as jnp
from jax.experimental import pallas as pl


def _body(x_ref, p_ref, o_ref):
    l = pl.program_id(0)
    BL = x_ref.shape[1]
    o_ref[...] = x_ref[...] + p_ref[pl.ds(l * BL, BL), :][None]


def kernel(x, pos_table):
    B, L, D = x.shape
    BL = 2048
    grid = (L // BL, B)
    return pl.pallas_call(
        _body,
        grid=grid,
        in_specs=[
            pl.BlockSpec((1, BL, D), lambda l, b: (b, l, 0)),
            pl.BlockSpec((L, D), lambda l, b: (0, 0)),
        ],
        out_specs=pl.BlockSpec((1, BL, D), lambda l, b: (b, l, 0)),
        out_shape=jax.ShapeDtypeStruct((B, L, D), x.dtype),
    )(x, pos_table)
